# baseline (speedup 1.0000x reference)
import functools

import jax
import jax.numpy as jnp
from jax import lax
from jax.experimental import pallas as pl
from jax.experimental.pallas import tpu as pltpu
from jax.experimental.pallas import tpu_sc as plsc

N = 9996
E = 159936
NP = 10240
DUMP = N
NTILES = 32
EPT = E // NTILES
G = 128
TPC = 40
RPT = NP // 16
BLK = 512
NB = 2
LAG = 1


def _make_prop(nch):
    mesh = plsc.VectorSubcoreMesh(core_axis_name="c", subcore_axis_name="s")
    d = 128

    @functools.partial(
        pl.kernel,
        out_type=jax.ShapeDtypeStruct((2, nch, NP, d), jnp.float32),
        mesh=mesh,
        scratch_types=[
            pltpu.VMEM((TPC, G), jnp.int32),
            pltpu.VMEM((TPC, G), jnp.int32),
            pltpu.VMEM((NB, G, d), jnp.float32),
            pltpu.VMEM((16, d), jnp.float32),
            pltpu.VMEM_SHARED((NP, d), jnp.float32),
            pltpu.SemaphoreType.DMA((NB,)),
            pltpu.SemaphoreType.DMA((NB,)),
        ],
    )
    def prop(hs, srcs, dsts, out, src_v, dst_v, stg, zb, acc, gsem, ssem):
        c = lax.axis_index("c")
        s = lax.axis_index("s")
        wid = c * 16 + s
        pltpu.sync_copy(srcs.at[wid], src_v)
        pltpu.sync_copy(dsts.at[wid], dst_v)
        z16 = jnp.zeros((16,), jnp.float32)
        for r in range(16):
            for cc in range(d // 16):
                zb[r, pl.ds(cc * 16, 16)] = z16
        row_lo = s * RPT

        for f in range(nch):
            def zero_body(j, carry):
                pltpu.sync_copy(zb, acc.at[pl.ds(row_lo + j * 16, 16)])
                return carry
            lax.fori_loop(0, RPT // 16, zero_body, 0)
            plsc.subcore_barrier()

            g_h = [None] * NB
            s_h = [None] * NB
            for j in range(TPC + LAG):
                b = j % NB
                if j < TPC:
                    if j >= NB:
                        s_h[b].wait()
                    g_h[b] = pltpu.async_copy(
                        hs.at[f].at[src_v.at[j]], stg.at[b], gsem.at[b])
                k = j - LAG
                if k >= 0:
                    bk = k % NB
                    g_h[bk].wait()
                    s_h[bk] = pltpu.async_copy(
                        stg.at[bk], acc.at[dst_v.at[k]], ssem.at[bk], add=True)
            for k in range(TPC - NB, TPC):
                s_h[k % NB].wait()

            plsc.subcore_barrier()
            pltpu.sync_copy(acc.at[pl.ds(row_lo, RPT)],
                            out.at[c, f, pl.ds(row_lo, RPT)])
            if f + 1 < nch:
                plsc.subcore_barrier()

    return prop


def _make_deg():
    mesh = plsc.VectorSubcoreMesh(core_axis_name="c", subcore_axis_name="s")
    DEPTH = 8

    @functools.partial(
        pl.kernel,
        out_type=jax.ShapeDtypeStruct((2, NP, 16), jnp.float32),
        mesh=mesh,
        scratch_types=[
            pltpu.VMEM((TPC, G), jnp.int32),
            pltpu.VMEM((G, 16), jnp.float32),
            pltpu.VMEM((32, 16), jnp.float32),
            pltpu.VMEM_SHARED((NP, 16), jnp.float32),
            pltpu.SemaphoreType.DMA,
        ],
    )
    def deg(dsts, out, dst_v, ob, zb, acc, ssem):
        c = lax.axis_index("c")
        s = lax.axis_index("s")
        wid = c * 16 + s
        pltpu.sync_copy(dsts.at[wid], dst_v)
        z16 = jnp.zeros((16,), jnp.float32)
        o16 = jnp.ones((16,), jnp.float32)
        for r in range(32):
            zb[r, pl.ds(0, 16)] = z16
        for r in range(G):
            ob[r, pl.ds(0, 16)] = o16
        row_lo = s * RPT

        def zero_body(j, carry):
            pltpu.sync_copy(zb, acc.at[pl.ds(row_lo + j * 32, 32)])
            return carry
        lax.fori_loop(0, RPT // 32, zero_body, 0)
        plsc.subcore_barrier()
        hs = [None] * TPC
        for j in range(TPC):
            if j >= DEPTH:
                hs[j - DEPTH].wait()
            hs[j] = pltpu.async_copy(ob, acc.at[dst_v.at[j]], ssem, add=True)
        for j in range(TPC - DEPTH, TPC):
            hs[j].wait()
        plsc.subcore_barrier()
        pltpu.sync_copy(acc.at[pl.ds(row_lo, RPT)], out.at[c, pl.ds(row_lo, RPT)])

    return deg


def _rowspec(shape_prefix, d):
    npre = len(shape_prefix)
    blk = tuple(shape_prefix) + (BLK, d)
    return pl.BlockSpec(blk, lambda i, n=npre: (0,) * n + (i, 0))


def _k0_body(dega_ref, x_ref, xs_ref, dinv_ref):
    i = pl.program_id(0)
    deg = dega_ref[0, :, 0:1] + dega_ref[1, :, 0:1] + 1.0
    rows = i * BLK + lax.broadcasted_iota(jnp.int32, (BLK, 1), 0)
    dinv = jnp.where(rows < N, lax.rsqrt(deg), 0.0)
    dinv_ref[...] = jnp.broadcast_to(dinv, (BLK, 128))
    xs_ref[0] = x_ref[...] * dinv


def _k1_body(s_ref, xs_ref, dinv_ref, w1_ref, b1_ref, w2_ref, out_ref):
    dinv = dinv_ref[:, 0:1]
    p0 = (s_ref[0, 0] + s_ref[1, 0] + xs_ref[0]) * dinv
    h1 = jnp.dot(p0, w1_ref[...], preferred_element_type=jnp.float32)
    h1 = jnp.maximum(h1 + b1_ref[...], 0.0)
    t2 = jnp.dot(h1, w2_ref[...], preferred_element_type=jnp.float32) * dinv
    for f in range(5):
        out_ref[f] = t2[:, f * 128:(f + 1) * 128]


def _make_kmid_body(nch_in, nch_out, dcol_out, zpad=0):
    def body(s_ref, hs_ref, dinv_ref, b_ref, w_ref, out_ref):
        dinv = dinv_ref[:, 0:1]
        cols = [s_ref[0, f] + s_ref[1, f] + hs_ref[f] for f in range(nch_in)]
        u = jnp.concatenate(cols, axis=1) if nch_in > 1 else cols[0]
        u = jnp.maximum(u * dinv + b_ref[...], 0.0)
        t = jnp.dot(u, w_ref[...], preferred_element_type=jnp.float32) * dinv
        if zpad:
            t = jnp.concatenate(
                [t, jnp.zeros((t.shape[0], zpad), jnp.float32)], axis=1)
        for f in range(nch_out):
            out_ref[f] = t[:, f * 128:(f + 1) * 128]
    return body


def _k5_body(s_ref, hs_ref, dinv_ref, b_ref, out_ref):
    dinv = dinv_ref[:, 0:1]
    u = (s_ref[0, 0] + s_ref[1, 0] + hs_ref[0]) * dinv
    out_ref[...] = jnp.maximum(u + b_ref[...], 0.0)


def _k6_body(h_ref, w_ref, b_ref, out_ref):
    out_ref[...] = jnp.dot(h_ref[...], w_ref[...],
                           preferred_element_type=jnp.float32) + b_ref[...]


def _whole(a):
    return pl.BlockSpec(a.shape, lambda i: (0,) * a.ndim)


def _tc(body, out_shape, in_specs, out_specs, args):
    return pl.pallas_call(
        body,
        grid=(NP // BLK,),
        in_specs=in_specs,
        out_specs=out_specs,
        out_shape=out_shape,
    )(*args)


def kernel(x, edge_index, W1, b1, W2, b2, W3, b3, W4, b4, W5, b5, fcW, fcb):
    f32 = jnp.float32
    x = x.astype(f32)
    xp = jnp.pad(x, ((0, NP - N), (0, 0)))
    order = jnp.argsort(edge_index[1])
    src = edge_index[0].astype(jnp.int32)[order].reshape(NTILES, EPT)
    dst = edge_index[1].astype(jnp.int32)[order].reshape(NTILES, EPT)
    pad = TPC * G - EPT
    srcs = jnp.pad(src, ((0, 0), (0, pad))).reshape(NTILES, TPC, G)
    dsts = jnp.pad(dst, ((0, 0), (0, pad)),
                   constant_values=DUMP).reshape(NTILES, TPC, G)

    w1p = jnp.pad(W1, ((0, 0), (0, 30)))
    b1p = jnp.pad(b1, (0, 30)).reshape(1, 1280)
    w2p = jnp.pad(W2, ((0, 30), (0, 15)))
    b2p = jnp.pad(b2, (0, 15)).reshape(1, 640)
    w3p = jnp.pad(W3, ((0, 15), (0, 0)))
    b3p = b3.reshape(1, 256)
    b4p = b4.reshape(1, 128)
    b5p = jnp.pad(b5, (0, 96)).reshape(1, 128)
    fcwp = jnp.pad(fcW, ((0, 0), (0, 124)))
    fcbp = jnp.pad(fcb, (0, 124)).reshape(1, 128)

    deg2 = _make_deg()(dsts)

    xs, dinvb = _tc(
        _k0_body,
        (jax.ShapeDtypeStruct((1, NP, 128), f32),
         jax.ShapeDtypeStruct((NP, 128), f32)),
        [_rowspec((2,), 16), _rowspec((), 128)],
        (_rowspec((1,), 128), _rowspec((), 128)),
        (deg2, xp))

    s0 = _make_prop(1)(xs, srcs, dsts)
    t2s = _tc(
        _k1_body,
        jax.ShapeDtypeStruct((5, NP, 128), f32),
        [_rowspec((2, 1), 128), _rowspec((1,), 128), _rowspec((), 128),
         _whole(w1p), _whole(b1p), _whole(w2p)],
        _rowspec((5,), 128),
        (s0, xs, dinvb, w1p, b1p, w2p))

    s2 = _make_prop(5)(t2s, srcs, dsts)
    t3s = _tc(
        _make_kmid_body(5, 2, 128),
        jax.ShapeDtypeStruct((2, NP, 128), f32),
        [_rowspec((2, 5), 128), _rowspec((5,), 128), _rowspec((), 128),
         _whole(b2p), _whole(w3p)],
        _rowspec((2,), 128),
        (s2, t2s, dinvb, b2p, w3p))

    s3 = _make_prop(2)(t3s, srcs, dsts)
    t4s = _tc(
        _make_kmid_body(2, 1, 128),
        jax.ShapeDtypeStruct((1, NP, 128), f32),
        [_rowspec((2, 2), 128), _rowspec((2,), 128), _rowspec((), 128),
         _whole(b3p), _whole(W4)],
        _rowspec((1,), 128),
        (s3, t3s, dinvb, b3p, W4))

    s4 = _make_prop(1)(t4s, srcs, dsts)
    t5s = _tc(
        _make_kmid_body(1, 1, 128, zpad=96),
        jax.ShapeDtypeStruct((1, NP, 128), f32),
        [_rowspec((2, 1), 128), _rowspec((1,), 128), _rowspec((), 128),
         _whole(b4p), _whole(W5)],
        _rowspec((1,), 128),
        (s4, t4s, dinvb, b4p, W5))

    s5 = _make_prop(1)(t5s, srcs, dsts)
    u5 = _tc(
        _k5_body,
        jax.ShapeDtypeStruct((NP, 128), f32),
        [_rowspec((2, 1), 128), _rowspec((1,), 128), _rowspec((), 128),
         _whole(b5p)],
        _rowspec((), 128),
        (s5, t5s, dinvb, b5p))

    h = u5[:N, :32].reshape(833, 384)
    hp = jnp.pad(h, ((0, 63), (0, 0)))
    out = pl.pallas_call(
        _k6_body,
        out_shape=jax.ShapeDtypeStruct((896, 128), f32),
    )(hp, fcwp, fcbp)
    return out[:833, :4]


# speedup vs baseline: 1.1159x; 1.1159x over previous
import functools

import jax
import jax.numpy as jnp
from jax import lax
from jax.experimental import pallas as pl
from jax.experimental.pallas import tpu as pltpu
from jax.experimental.pallas import tpu_sc as plsc

N = 9996
E = 159936
NP = 10240
DUMP = N
NTILES = 32
EPT = E // NTILES
G = 128
TPC = 40
RPT = NP // 16
BLK = 512
NB = 2
LAG = 1


def _make_prop(nch):
    mesh = plsc.VectorSubcoreMesh(core_axis_name="c", subcore_axis_name="s")
    d = 128

    @functools.partial(
        pl.kernel,
        out_type=jax.ShapeDtypeStruct((2, nch, NP, d), jnp.float32),
        mesh=mesh,
        scratch_types=[
            pltpu.VMEM((TPC, G), jnp.int32),
            pltpu.VMEM((TPC, G), jnp.int32),
            pltpu.VMEM((NB, G, d), jnp.float32),
            pltpu.VMEM((16, d), jnp.float32),
            pltpu.VMEM_SHARED((NP, d), jnp.float32),
            pltpu.SemaphoreType.DMA((NB,)),
            pltpu.SemaphoreType.DMA((NB,)),
        ],
    )
    def prop(hs, srcs, dsts, out, src_v, dst_v, stg, zb, acc, gsem, ssem):
        c = lax.axis_index("c")
        s = lax.axis_index("s")
        wid = c * 16 + s
        pltpu.sync_copy(srcs.at[wid], src_v)
        pltpu.sync_copy(dsts.at[wid], dst_v)
        z16 = jnp.zeros((16,), jnp.float32)
        for r in range(16):
            for cc in range(d // 16):
                zb[r, pl.ds(cc * 16, 16)] = z16
        row_lo = s * RPT

        for f in range(nch):
            def zero_body(j, carry):
                pltpu.sync_copy(zb, acc.at[pl.ds(row_lo + j * 16, 16)])
                return carry
            lax.fori_loop(0, RPT // 16, zero_body, 0)
            plsc.subcore_barrier()

            g_h = [None] * NB
            s_h = [None] * NB
            for j in range(TPC + LAG):
                b = j % NB
                if j < TPC:
                    if j >= NB:
                        s_h[b].wait()
                    g_h[b] = pltpu.async_copy(
                        hs.at[f].at[src_v.at[j]], stg.at[b], gsem.at[b])
                k = j - LAG
                if k >= 0:
                    bk = k % NB
                    g_h[bk].wait()
                    s_h[bk] = pltpu.async_copy(
                        stg.at[bk], acc.at[dst_v.at[k]], ssem.at[bk], add=True)
            for k in range(TPC - NB, TPC):
                s_h[k % NB].wait()

            plsc.subcore_barrier()
            pltpu.sync_copy(acc.at[pl.ds(row_lo, RPT)],
                            out.at[c, f, pl.ds(row_lo, RPT)])
            if f + 1 < nch:
                plsc.subcore_barrier()

    return prop


def _make_deg():
    mesh = plsc.VectorSubcoreMesh(core_axis_name="c", subcore_axis_name="s")
    DEPTH = 8

    @functools.partial(
        pl.kernel,
        out_type=jax.ShapeDtypeStruct((2, NP, 16), jnp.float32),
        mesh=mesh,
        scratch_types=[
            pltpu.VMEM((TPC, G), jnp.int32),
            pltpu.VMEM((G, 16), jnp.float32),
            pltpu.VMEM((32, 16), jnp.float32),
            pltpu.VMEM_SHARED((NP, 16), jnp.float32),
            pltpu.SemaphoreType.DMA,
        ],
    )
    def deg(dsts, out, dst_v, ob, zb, acc, ssem):
        c = lax.axis_index("c")
        s = lax.axis_index("s")
        wid = c * 16 + s
        pltpu.sync_copy(dsts.at[wid], dst_v)
        z16 = jnp.zeros((16,), jnp.float32)
        o16 = jnp.ones((16,), jnp.float32)
        for r in range(32):
            zb[r, pl.ds(0, 16)] = z16
        for r in range(G):
            ob[r, pl.ds(0, 16)] = o16
        row_lo = s * RPT

        def zero_body(j, carry):
            pltpu.sync_copy(zb, acc.at[pl.ds(row_lo + j * 32, 32)])
            return carry
        lax.fori_loop(0, RPT // 32, zero_body, 0)
        plsc.subcore_barrier()
        hs = [None] * TPC
        for j in range(TPC):
            if j >= DEPTH:
                hs[j - DEPTH].wait()
            hs[j] = pltpu.async_copy(ob, acc.at[dst_v.at[j]], ssem, add=True)
        for j in range(TPC - DEPTH, TPC):
            hs[j].wait()
        plsc.subcore_barrier()
        pltpu.sync_copy(acc.at[pl.ds(row_lo, RPT)], out.at[c, pl.ds(row_lo, RPT)])

    return deg


def _rowspec(shape_prefix, d):
    npre = len(shape_prefix)
    blk = tuple(shape_prefix) + (BLK, d)
    return pl.BlockSpec(blk, lambda i, n=npre: (0,) * n + (i, 0))


def _k0_body(dega_ref, x_ref, xs_ref, dinv_ref):
    i = pl.program_id(0)
    deg = dega_ref[0, :, 0:1] + dega_ref[1, :, 0:1] + 1.0
    rows = i * BLK + lax.broadcasted_iota(jnp.int32, (BLK, 1), 0)
    dinv = jnp.where(rows < N, lax.rsqrt(deg), 0.0)
    dinv_ref[...] = jnp.broadcast_to(dinv, (BLK, 128))
    xs_ref[0] = x_ref[...] * dinv


def _k1_body(s_ref, xs_ref, dinv_ref, w1_ref, b1_ref, w2_ref, out_ref):
    dinv = dinv_ref[:, 0:1]
    p0 = (s_ref[0, 0] + s_ref[1, 0] + xs_ref[0]) * dinv
    h1 = jnp.dot(p0, w1_ref[...], preferred_element_type=jnp.float32)
    h1 = jnp.maximum(h1 + b1_ref[...], 0.0)
    t2 = jnp.dot(h1, w2_ref[...], preferred_element_type=jnp.float32) * dinv
    for f in range(5):
        out_ref[f] = t2[:, f * 128:(f + 1) * 128]


def _make_kmid_body(nch_in, nch_out, dcol_out, zpad=0):
    def body(s_ref, hs_ref, dinv_ref, b_ref, w_ref, out_ref):
        dinv = dinv_ref[:, 0:1]
        cols = [s_ref[0, f] + s_ref[1, f] + hs_ref[f] for f in range(nch_in)]
        u = jnp.concatenate(cols, axis=1) if nch_in > 1 else cols[0]
        u = jnp.maximum(u * dinv + b_ref[...], 0.0)
        t = jnp.dot(u, w_ref[...], preferred_element_type=jnp.float32) * dinv
        if zpad:
            t = jnp.concatenate(
                [t, jnp.zeros((t.shape[0], zpad), jnp.float32)], axis=1)
        for f in range(nch_out):
            out_ref[f] = t[:, f * 128:(f + 1) * 128]
    return body


def _k5_body(s_ref, hs_ref, dinv_ref, b_ref, out_ref):
    dinv = dinv_ref[:, 0:1]
    u = (s_ref[0, 0] + s_ref[1, 0] + hs_ref[0]) * dinv
    out_ref[...] = jnp.maximum(u + b_ref[...], 0.0)


def _k6_body(h_ref, w_ref, b_ref, out_ref):
    out_ref[...] = jnp.dot(h_ref[...], w_ref[...],
                           preferred_element_type=jnp.float32) + b_ref[...]


def _whole(a):
    return pl.BlockSpec(a.shape, lambda i: (0,) * a.ndim)


def _tc(body, out_shape, in_specs, out_specs, args):
    return pl.pallas_call(
        body,
        grid=(NP // BLK,),
        in_specs=in_specs,
        out_specs=out_specs,
        out_shape=out_shape,
    )(*args)


def kernel(x, edge_index, W1, b1, W2, b2, W3, b3, W4, b4, W5, b5, fcW, fcb):
    f32 = jnp.float32
    x = x.astype(f32)
    xp = jnp.pad(x, ((0, NP - N), (0, 0)))
    src = edge_index[0].astype(jnp.int32).reshape(NTILES, EPT)
    dst = edge_index[1].astype(jnp.int32).reshape(NTILES, EPT)
    pad = TPC * G - EPT
    srcs = jnp.pad(src, ((0, 0), (0, pad))).reshape(NTILES, TPC, G)
    dsts = jnp.pad(dst, ((0, 0), (0, pad)),
                   constant_values=DUMP).reshape(NTILES, TPC, G)

    w1p = jnp.pad(W1, ((0, 0), (0, 30)))
    b1p = jnp.pad(b1, (0, 30)).reshape(1, 1280)
    w2p = jnp.pad(W2, ((0, 30), (0, 15)))
    b2p = jnp.pad(b2, (0, 15)).reshape(1, 640)
    w3p = jnp.pad(W3, ((0, 15), (0, 0)))
    b3p = b3.reshape(1, 256)
    b4p = b4.reshape(1, 128)
    b5p = jnp.pad(b5, (0, 96)).reshape(1, 128)
    fcwp = jnp.pad(fcW, ((0, 0), (0, 124)))
    fcbp = jnp.pad(fcb, (0, 124)).reshape(1, 128)

    deg2 = _make_deg()(dsts)

    xs, dinvb = _tc(
        _k0_body,
        (jax.ShapeDtypeStruct((1, NP, 128), f32),
         jax.ShapeDtypeStruct((NP, 128), f32)),
        [_rowspec((2,), 16), _rowspec((), 128)],
        (_rowspec((1,), 128), _rowspec((), 128)),
        (deg2, xp))

    s0 = _make_prop(1)(xs, srcs, dsts)
    t2s = _tc(
        _k1_body,
        jax.ShapeDtypeStruct((5, NP, 128), f32),
        [_rowspec((2, 1), 128), _rowspec((1,), 128), _rowspec((), 128),
         _whole(w1p), _whole(b1p), _whole(w2p)],
        _rowspec((5,), 128),
        (s0, xs, dinvb, w1p, b1p, w2p))

    s2 = _make_prop(5)(t2s, srcs, dsts)
    t3s = _tc(
        _make_kmid_body(5, 2, 128),
        jax.ShapeDtypeStruct((2, NP, 128), f32),
        [_rowspec((2, 5), 128), _rowspec((5,), 128), _rowspec((), 128),
         _whole(b2p), _whole(w3p)],
        _rowspec((2,), 128),
        (s2, t2s, dinvb, b2p, w3p))

    s3 = _make_prop(2)(t3s, srcs, dsts)
    t4s = _tc(
        _make_kmid_body(2, 1, 128),
        jax.ShapeDtypeStruct((1, NP, 128), f32),
        [_rowspec((2, 2), 128), _rowspec((2,), 128), _rowspec((), 128),
         _whole(b3p), _whole(W4)],
        _rowspec((1,), 128),
        (s3, t3s, dinvb, b3p, W4))

    s4 = _make_prop(1)(t4s, srcs, dsts)
    t5s = _tc(
        _make_kmid_body(1, 1, 128, zpad=96),
        jax.ShapeDtypeStruct((1, NP, 128), f32),
        [_rowspec((2, 1), 128), _rowspec((1,), 128), _rowspec((), 128),
         _whole(b4p), _whole(W5)],
        _rowspec((1,), 128),
        (s4, t4s, dinvb, b4p, W5))

    s5 = _make_prop(1)(t5s, srcs, dsts)
    u5 = _tc(
        _k5_body,
        jax.ShapeDtypeStruct((NP, 128), f32),
        [_rowspec((2, 1), 128), _rowspec((1,), 128), _rowspec((), 128),
         _whole(b5p)],
        _rowspec((), 128),
        (s5, t5s, dinvb, b5p))

    h = u5[:N, :32].reshape(833, 384)
    hp = jnp.pad(h, ((0, 63), (0, 0)))
    out = pl.pallas_call(
        _k6_body,
        out_shape=jax.ShapeDtypeStruct((896, 128), f32),
    )(hp, fcwp, fcbp)
    return out[:833, :4]


# speedup vs baseline: 3.1163x; 2.7927x over previous
import functools

import jax
import jax.numpy as jnp
from jax import lax
from jax.experimental import pallas as pl
from jax.experimental.pallas import tpu as pltpu
from jax.experimental.pallas import tpu_sc as plsc

N = 9996
E = 159936
NP = 10240
DUMP = N
NTILES = 32
EPT = E // NTILES
G = 128
TPC = 39
GM = 8
RPT = NP // 16
BLK = 512
NB = 2
LAG = 1


def _make_prop(nch):
    mesh = plsc.VectorSubcoreMesh(core_axis_name="c", subcore_axis_name="s")
    d = 128

    @functools.partial(
        pl.kernel,
        out_type=jax.ShapeDtypeStruct((2, nch, NP, d), jnp.float32),
        mesh=mesh,
        scratch_types=[
            pltpu.VMEM((TPC, G), jnp.int32),
            pltpu.VMEM((TPC, G), jnp.int32),
            pltpu.VMEM((GM,), jnp.int32),
            pltpu.VMEM((GM,), jnp.int32),
            pltpu.VMEM((NB, G, d), jnp.float32),
            pltpu.VMEM((GM, d), jnp.float32),
            pltpu.VMEM((16, d), jnp.float32),
            pltpu.VMEM_SHARED((NP, d), jnp.float32),
            pltpu.SemaphoreType.DMA((NB,)),
            pltpu.SemaphoreType.DMA((NB,)),
        ],
    )
    def prop(hs, srcs, dsts, srcm, dstm, out,
             src_v, dst_v, srcm_v, dstm_v, stg, stgm, zb, acc, gsem, ssem):
        c = lax.axis_index("c")
        s = lax.axis_index("s")
        wid = c * 16 + s
        pltpu.sync_copy(srcs.at[wid], src_v)
        pltpu.sync_copy(dsts.at[wid], dst_v)
        pltpu.sync_copy(srcm.at[wid], srcm_v)
        pltpu.sync_copy(dstm.at[wid], dstm_v)
        z16 = jnp.zeros((16,), jnp.float32)
        for r in range(16):
            for cc in range(d // 16):
                zb[r, pl.ds(cc * 16, 16)] = z16
        row_lo = s * RPT

        for f in range(nch):
            def zero_body(j, carry):
                pltpu.sync_copy(zb, acc.at[pl.ds(row_lo + j * 16, 16)])
                return carry
            lax.fori_loop(0, RPT // 16, zero_body, 0)
            plsc.subcore_barrier()

            g_h = [None] * NB
            s_h = [None] * NB
            for j in range(TPC + LAG):
                b = j % NB
                if j < TPC:
                    if j >= NB:
                        s_h[b].wait()
                    g_h[b] = pltpu.async_copy(
                        hs.at[f].at[src_v.at[j]], stg.at[b], gsem.at[b])
                k = j - LAG
                if k >= 0:
                    bk = k % NB
                    g_h[bk].wait()
                    s_h[bk] = pltpu.async_copy(
                        stg.at[bk], acc.at[dst_v.at[k]], ssem.at[bk], add=True)
            pltpu.async_copy(hs.at[f].at[srcm_v], stgm, gsem.at[0]).wait()
            pltpu.sync_copy(stgm, acc.at[dstm_v], add=True)
            for k in range(TPC - NB, TPC):
                s_h[k % NB].wait()

            plsc.subcore_barrier()
            pltpu.sync_copy(acc.at[pl.ds(row_lo, RPT)],
                            out.at[c, f, pl.ds(row_lo, RPT)])
            if f + 1 < nch:
                plsc.subcore_barrier()

    return prop


def _make_deg():
    mesh = plsc.VectorSubcoreMesh(core_axis_name="c", subcore_axis_name="s")
    DEPTH = 8

    @functools.partial(
        pl.kernel,
        out_type=jax.ShapeDtypeStruct((2, NP, 16), jnp.float32),
        mesh=mesh,
        scratch_types=[
            pltpu.VMEM((TPC, G), jnp.int32),
            pltpu.VMEM((GM,), jnp.int32),
            pltpu.VMEM((G, 16), jnp.float32),
            pltpu.VMEM((32, 16), jnp.float32),
            pltpu.VMEM_SHARED((NP, 16), jnp.float32),
            pltpu.SemaphoreType.DMA,
        ],
    )
    def deg(dsts, dstm, out, dst_v, dstm_v, ob, zb, acc, ssem):
        c = lax.axis_index("c")
        s = lax.axis_index("s")
        wid = c * 16 + s
        pltpu.sync_copy(dsts.at[wid], dst_v)
        pltpu.sync_copy(dstm.at[wid], dstm_v)
        z16 = jnp.zeros((16,), jnp.float32)
        o16 = jnp.ones((16,), jnp.float32)
        for r in range(32):
            zb[r, pl.ds(0, 16)] = z16
        for r in range(G):
            ob[r, pl.ds(0, 16)] = o16
        row_lo = s * RPT

        def zero_body(j, carry):
            pltpu.sync_copy(zb, acc.at[pl.ds(row_lo + j * 32, 32)])
            return carry
        lax.fori_loop(0, RPT // 32, zero_body, 0)
        plsc.subcore_barrier()
        hs = [None] * TPC
        for j in range(TPC):
            if j >= DEPTH:
                hs[j - DEPTH].wait()
            hs[j] = pltpu.async_copy(ob, acc.at[dst_v.at[j]], ssem, add=True)
        for j in range(TPC - DEPTH, TPC):
            hs[j].wait()
        pltpu.sync_copy(ob.at[pl.ds(0, GM)], acc.at[dstm_v], add=True)
        plsc.subcore_barrier()
        pltpu.sync_copy(acc.at[pl.ds(row_lo, RPT)], out.at[c, pl.ds(row_lo, RPT)])

    return deg


def _rowspec(shape_prefix, d):
    npre = len(shape_prefix)
    blk = tuple(shape_prefix) + (BLK, d)
    return pl.BlockSpec(blk, lambda i, n=npre: (0,) * n + (i, 0))


def _k0_body(dega_ref, x_ref, xs_ref, dinv_ref):
    i = pl.program_id(0)
    deg = dega_ref[0, :, 0:1] + dega_ref[1, :, 0:1] + 1.0
    rows = i * BLK + lax.broadcasted_iota(jnp.int32, (BLK, 1), 0)
    dinv = jnp.where(rows < N, lax.rsqrt(deg), 0.0)
    dinv_ref[...] = jnp.broadcast_to(dinv, (BLK, 128))
    xs_ref[0] = x_ref[...] * dinv


def _k1_body(s_ref, xs_ref, dinv_ref, w1_ref, b1_ref, w2_ref, out_ref):
    dinv = dinv_ref[:, 0:1]
    p0 = (s_ref[0, 0] + s_ref[1, 0] + xs_ref[0]) * dinv
    h1 = jnp.dot(p0, w1_ref[...], preferred_element_type=jnp.float32)
    h1 = jnp.maximum(h1 + b1_ref[...], 0.0)
    t2 = jnp.dot(h1, w2_ref[...], preferred_element_type=jnp.float32) * dinv
    for f in range(5):
        out_ref[f] = t2[:, f * 128:(f + 1) * 128]


def _make_kmid_body(nch_in, nch_out, dcol_out, zpad=0):
    def body(s_ref, hs_ref, dinv_ref, b_ref, w_ref, out_ref):
        dinv = dinv_ref[:, 0:1]
        cols = [s_ref[0, f] + s_ref[1, f] + hs_ref[f] for f in range(nch_in)]
        u = jnp.concatenate(cols, axis=1) if nch_in > 1 else cols[0]
        u = jnp.maximum(u * dinv + b_ref[...], 0.0)
        t = jnp.dot(u, w_ref[...], preferred_element_type=jnp.float32) * dinv
        if zpad:
            t = jnp.concatenate(
                [t, jnp.zeros((t.shape[0], zpad), jnp.float32)], axis=1)
        for f in range(nch_out):
            out_ref[f] = t[:, f * 128:(f + 1) * 128]
    return body


def _k5_body(s_ref, hs_ref, dinv_ref, b_ref, out_ref):
    dinv = dinv_ref[:, 0:1]
    u = (s_ref[0, 0] + s_ref[1, 0] + hs_ref[0]) * dinv
    out_ref[...] = jnp.maximum(u + b_ref[...], 0.0)


def _k6_body(h_ref, w_ref, b_ref, out_ref):
    out_ref[...] = jnp.dot(h_ref[...], w_ref[...],
                           preferred_element_type=jnp.float32) + b_ref[...]


def _whole(a):
    return pl.BlockSpec(a.shape, lambda i: (0,) * a.ndim)


def _tc(body, out_shape, in_specs, out_specs, args):
    return pl.pallas_call(
        body,
        grid=(NP // BLK,),
        in_specs=in_specs,
        out_specs=out_specs,
        out_shape=out_shape,
    )(*args)


def kernel(x, edge_index, W1, b1, W2, b2, W3, b3, W4, b4, W5, b5, fcW, fcb):
    f32 = jnp.float32
    x = x.astype(f32)
    xp = jnp.pad(x, ((0, NP - N), (0, 0)))
    src = edge_index[0].astype(jnp.int32).reshape(NTILES, EPT)
    dst = edge_index[1].astype(jnp.int32).reshape(NTILES, EPT)
    srcs = src[:, :TPC * G].reshape(NTILES, TPC, G)
    dsts = dst[:, :TPC * G].reshape(NTILES, TPC, G)
    mpad = GM - (EPT - TPC * G)
    srcm = jnp.pad(src[:, TPC * G:], ((0, 0), (0, mpad)))
    dstm = jnp.pad(dst[:, TPC * G:], ((0, 0), (0, mpad)),
                   constant_values=DUMP)

    w1p = jnp.pad(W1, ((0, 0), (0, 30)))
    b1p = jnp.pad(b1, (0, 30)).reshape(1, 1280)
    w2p = jnp.pad(W2, ((0, 30), (0, 15)))
    b2p = jnp.pad(b2, (0, 15)).reshape(1, 640)
    w3p = jnp.pad(W3, ((0, 15), (0, 0)))
    b3p = b3.reshape(1, 256)
    b4p = b4.reshape(1, 128)
    b5p = jnp.pad(b5, (0, 96)).reshape(1, 128)
    fcwp = jnp.pad(fcW, ((0, 0), (0, 124)))
    fcbp = jnp.pad(fcb, (0, 124)).reshape(1, 128)

    deg2 = _make_deg()(dsts, dstm)

    xs, dinvb = _tc(
        _k0_body,
        (jax.ShapeDtypeStruct((1, NP, 128), f32),
         jax.ShapeDtypeStruct((NP, 128), f32)),
        [_rowspec((2,), 16), _rowspec((), 128)],
        (_rowspec((1,), 128), _rowspec((), 128)),
        (deg2, xp))

    s0 = _make_prop(1)(xs, srcs, dsts, srcm, dstm)
    t2s = _tc(
        _k1_body,
        jax.ShapeDtypeStruct((5, NP, 128), f32),
        [_rowspec((2, 1), 128), _rowspec((1,), 128), _rowspec((), 128),
         _whole(w1p), _whole(b1p), _whole(w2p)],
        _rowspec((5,), 128),
        (s0, xs, dinvb, w1p, b1p, w2p))

    s2 = _make_prop(5)(t2s, srcs, dsts, srcm, dstm)
    t3s = _tc(
        _make_kmid_body(5, 2, 128),
        jax.ShapeDtypeStruct((2, NP, 128), f32),
        [_rowspec((2, 5), 128), _rowspec((5,), 128), _rowspec((), 128),
         _whole(b2p), _whole(w3p)],
        _rowspec((2,), 128),
        (s2, t2s, dinvb, b2p, w3p))

    s3 = _make_prop(2)(t3s, srcs, dsts, srcm, dstm)
    t4s = _tc(
        _make_kmid_body(2, 1, 128),
        jax.ShapeDtypeStruct((1, NP, 128), f32),
        [_rowspec((2, 2), 128), _rowspec((2,), 128), _rowspec((), 128),
         _whole(b3p), _whole(W4)],
        _rowspec((1,), 128),
        (s3, t3s, dinvb, b3p, W4))

    s4 = _make_prop(1)(t4s, srcs, dsts, srcm, dstm)
    t5s = _tc(
        _make_kmid_body(1, 1, 128, zpad=96),
        jax.ShapeDtypeStruct((1, NP, 128), f32),
        [_rowspec((2, 1), 128), _rowspec((1,), 128), _rowspec((), 128),
         _whole(b4p), _whole(W5)],
        _rowspec((1,), 128),
        (s4, t4s, dinvb, b4p, W5))

    s5 = _make_prop(1)(t5s, srcs, dsts, srcm, dstm)
    u5 = _tc(
        _k5_body,
        jax.ShapeDtypeStruct((NP, 128), f32),
        [_rowspec((2, 1), 128), _rowspec((1,), 128), _rowspec((), 128),
         _whole(b5p)],
        _rowspec((), 128),
        (s5, t5s, dinvb, b5p))

    h = u5[:N, :32].reshape(833, 384)
    hp = jnp.pad(h, ((0, 63), (0, 0)))
    out = pl.pallas_call(
        _k6_body,
        out_shape=jax.ShapeDtypeStruct((896, 128), f32),
    )(hp, fcwp, fcbp)
    return out[:833, :4]
